# baseline (speedup 1.0000x reference)
import functools

import jax
import jax.numpy as jnp
from jax.experimental import pallas as pl
from jax.experimental.pallas import tpu as pltpu
from jax.experimental.pallas import tpu_sc as plsc

N = 2048
D = 768
H = 768
E = 8

BE = 256
PADN = 6144
NBLK = PADN // BE
BG = 256
NBG = N // BG
NC, NS = 2, 16
NW = NC * NS
BPW = N // NW


def _top2(logits):
    neg_inf = jnp.float32(-jnp.inf)
    iota = jax.lax.broadcasted_iota(jnp.int32, logits.shape, 1)
    m1 = jnp.max(logits, axis=-1, keepdims=True)
    eq1 = logits == m1
    i1 = jnp.min(jnp.where(eq1, iota, E), axis=-1, keepdims=True)
    first1 = iota == i1
    l2 = jnp.where(first1, neg_inf, logits)
    m2 = jnp.max(l2, axis=-1, keepdims=True)
    eq2 = l2 == m2
    i2 = jnp.min(jnp.where(eq2, iota, E), axis=-1, keepdims=True)
    first2 = iota == i2
    sel = first1 | first2
    wsel = jnp.where(sel, jnp.exp(logits - m1), 0.0)
    wsum = jnp.sum(wsel, axis=-1, keepdims=True)
    return first1, first2, sel, wsel, wsum


def _gate_body(x_ref, wg_ref, bg_ref, d0_ref, d1_ref, w01_ref, eid_ref,
               counts_s, po_s, run_s):
    p = pl.program_id(0)
    i = pl.program_id(1)
    xb = x_ref[...]
    logits = jnp.dot(xb, wg_ref[...], preferred_element_type=jnp.float32)
    logits = logits + bg_ref[...]
    first1, first2, sel, wsel, wsum = _top2(logits)
    sel_f = sel.astype(jnp.float32)

    @pl.when((p == 0) & (i == 0))
    def _():
        counts_s[...] = jnp.zeros((1, E), jnp.float32)

    @pl.when(p == 0)
    def _():
        counts_s[...] += jnp.sum(sel_f, axis=0, keepdims=True)

    @pl.when((p == 1) & (i == 0))
    def _():
        c = counts_s[...]
        cpad = jnp.floor((c + (BE - 1)) / BE) * BE
        lt = (jax.lax.broadcasted_iota(jnp.int32, (E, E), 0)
              < jax.lax.broadcasted_iota(jnp.int32, (E, E), 1)
              ).astype(jnp.bfloat16)
        po_s[...] = jnp.dot(cpad.astype(jnp.bfloat16), lt,
                            preferred_element_type=jnp.float32)
        run_s[...] = jnp.zeros((1, E), jnp.float32)

    @pl.when(p == 1)
    def _():
        po = po_s[...]
        run = run_s[...]
        ltn = (jax.lax.broadcasted_iota(jnp.int32, (BG, BG), 1)
               < jax.lax.broadcasted_iota(jnp.int32, (BG, BG), 0)
               ).astype(jnp.bfloat16)
        pos = run + jnp.dot(ltn, sel_f.astype(jnp.bfloat16),
                            preferred_element_type=jnp.float32)
        run_s[...] = run + jnp.sum(sel_f, axis=0, keepdims=True)
        dest = po + pos
        d0 = jnp.sum(jnp.where(first1, dest, 0.0), axis=-1, keepdims=True)
        d1 = jnp.sum(jnp.where(first2, dest, 0.0), axis=-1, keepdims=True)
        w0 = jnp.sum(jnp.where(first1, wsel, 0.0), axis=-1, keepdims=True) / wsum
        w1 = jnp.sum(jnp.where(first2, wsel, 0.0), axis=-1, keepdims=True) / wsum
        col = jax.lax.broadcasted_iota(jnp.int32, (BG, E), 1)
        d0_ref[...] = jnp.reshape(d0.astype(jnp.int32), (1, BG))
        d1_ref[...] = jnp.reshape(d1.astype(jnp.int32), (1, BG))
        w01_ref[...] = (jnp.where(col == 0, w0, 0.0)
                        + jnp.where(col == 1, w1, 0.0))
        bst = jnp.float32(BE) * jax.lax.broadcasted_iota(
            jnp.int32, (E, NBLK), 1).astype(jnp.float32)
        m = (jnp.reshape(po, (E, 1)) <= bst).astype(jnp.int32)
        eid_ref[...] = jnp.sum(m, axis=0, keepdims=True) - 1


def _gate(x, Wg, bg):
    return pl.pallas_call(
        _gate_body,
        grid=(2, NBG),
        in_specs=[
            pl.BlockSpec((BG, D), lambda p, i: (i, 0)),
            pl.BlockSpec((D, E), lambda p, i: (0, 0)),
            pl.BlockSpec((1, E), lambda p, i: (0, 0)),
        ],
        out_specs=[
            pl.BlockSpec((1, BG), lambda p, i: (0, i)),
            pl.BlockSpec((1, BG), lambda p, i: (0, i)),
            pl.BlockSpec((BG, E), lambda p, i: (i, 0)),
            pl.BlockSpec((1, NBLK), lambda p, i: (0, 0)),
        ],
        out_shape=[
            jax.ShapeDtypeStruct((1, N), jnp.int32),
            jax.ShapeDtypeStruct((1, N), jnp.int32),
            jax.ShapeDtypeStruct((N, E), jnp.float32),
            jax.ShapeDtypeStruct((1, NBLK), jnp.int32),
        ],
        scratch_shapes=[
            pltpu.VMEM((1, E), jnp.float32),
            pltpu.VMEM((1, E), jnp.float32),
            pltpu.VMEM((1, E), jnp.float32),
        ],
    )(x, Wg, bg.reshape(1, E))


def _sc_dispatch(x, d0, d1):
    mesh = plsc.VectorSubcoreMesh(core_axis_name="c", subcore_axis_name="s")

    @functools.partial(
        pl.kernel, mesh=mesh,
        out_type=jax.ShapeDtypeStruct((PADN, D // 2), jnp.int32),
        scratch_types=[
            pltpu.VMEM((BPW,), jnp.int32),
            pltpu.VMEM((BPW,), jnp.int32),
            pltpu.VMEM((BPW, D // 2), jnp.int32),
            pltpu.SemaphoreType.DMA,
        ],
    )
    def disp(x_hbm, d0_hbm, d1_hbm, xr_hbm, i0_v, i1_v, rows_v, sem):
        wid = jax.lax.axis_index("s") * NC + jax.lax.axis_index("c")
        base = wid * BPW
        pltpu.sync_copy(d0_hbm.at[pl.ds(base, BPW)], i0_v)
        pltpu.sync_copy(d1_hbm.at[pl.ds(base, BPW)], i1_v)
        pltpu.sync_copy(x_hbm.at[pl.ds(base, BPW)], rows_v)
        pltpu.async_copy(rows_v, xr_hbm.at[i0_v], sem).wait()
        pltpu.async_copy(rows_v, xr_hbm.at[i1_v], sem).wait()

    return disp(x, d0, d1)


def _sc_gather(y, d0, d1):
    mesh = plsc.VectorSubcoreMesh(core_axis_name="c", subcore_axis_name="s")

    @functools.partial(
        pl.kernel, mesh=mesh,
        out_type=[jax.ShapeDtypeStruct((N, H // 2), jnp.int32),
                  jax.ShapeDtypeStruct((N, H // 2), jnp.int32)],
        scratch_types=[
            pltpu.VMEM((BPW,), jnp.int32),
            pltpu.VMEM((BPW, H // 2), jnp.int32),
            pltpu.SemaphoreType.DMA,
        ],
    )
    def gath(y_hbm, d0_hbm, d1_hbm, o0_hbm, o1_hbm, i_v, rows_v, sem):
        wid = jax.lax.axis_index("s") * NC + jax.lax.axis_index("c")
        base = wid * BPW
        pltpu.sync_copy(d0_hbm.at[pl.ds(base, BPW)], i_v)
        pltpu.async_copy(y_hbm.at[i_v], rows_v, sem).wait()
        pltpu.sync_copy(rows_v, o0_hbm.at[pl.ds(base, BPW)])
        pltpu.sync_copy(d1_hbm.at[pl.ds(base, BPW)], i_v)
        pltpu.async_copy(y_hbm.at[i_v], rows_v, sem).wait()
        pltpu.sync_copy(rows_v, o1_hbm.at[pl.ds(base, BPW)])

    return gath(y, d0, d1)


def _expert_body(eid_ref, xr_ref, w1_ref, b1_ref, g1_ref, be1_ref,
                 w2_ref, b2_ref, y_ref):
    xb16 = xr_ref[...]
    h = jnp.dot(xb16, w1_ref[0], preferred_element_type=jnp.float32)
    h = h + b1_ref[0]
    mu = jnp.mean(h, axis=-1, keepdims=True)
    var = jnp.mean((h - mu) ** 2, axis=-1, keepdims=True)
    h = (h - mu) * jax.lax.rsqrt(var + 1e-5)
    h = h * g1_ref[0] + be1_ref[0]
    h = h * 0.5 * (1.0 + jax.lax.erf(h * jnp.float32(0.7071067811865476)))
    y = jnp.dot(h.astype(jnp.bfloat16), w2_ref[0],
                preferred_element_type=jnp.float32)
    y_ref[...] = (y + b2_ref[0]).astype(jnp.bfloat16)


def _expert(eid, xr, w1b, b1, g1, be1, w2b, b2):
    grid_spec = pltpu.PrefetchScalarGridSpec(
        num_scalar_prefetch=1,
        grid=(NBLK,),
        in_specs=[
            pl.BlockSpec((BE, D), lambda i, eid_ref: (i, 0)),
            pl.BlockSpec((1, D, H), lambda i, eid_ref: (eid_ref[i], 0, 0)),
            pl.BlockSpec((1, 1, H), lambda i, eid_ref: (eid_ref[i], 0, 0)),
            pl.BlockSpec((1, 1, H), lambda i, eid_ref: (eid_ref[i], 0, 0)),
            pl.BlockSpec((1, 1, H), lambda i, eid_ref: (eid_ref[i], 0, 0)),
            pl.BlockSpec((1, D, H), lambda i, eid_ref: (eid_ref[i], 0, 0)),
            pl.BlockSpec((1, 1, H), lambda i, eid_ref: (eid_ref[i], 0, 0)),
        ],
        out_specs=pl.BlockSpec((BE, H), lambda i, eid_ref: (i, 0)),
    )
    return pl.pallas_call(
        _expert_body,
        grid_spec=grid_spec,
        out_shape=jax.ShapeDtypeStruct((PADN, H), jnp.bfloat16),
    )(eid, xr, w1b, b1.reshape(E, 1, H), g1.reshape(E, 1, H),
      be1.reshape(E, 1, H), w2b, b2.reshape(E, 1, H))


def _combine_body(yg0_ref, yg1_ref, w01_ref, out_ref):
    w = w01_ref[...]
    out_ref[...] = (yg0_ref[...].astype(jnp.float32) * w[:, 0:1]
                    + yg1_ref[...].astype(jnp.float32) * w[:, 1:2])


def _combine(yg0, yg1, w01):
    return pl.pallas_call(
        _combine_body,
        grid=(NBG,),
        in_specs=[
            pl.BlockSpec((BG, H), lambda i: (i, 0)),
            pl.BlockSpec((BG, H), lambda i: (i, 0)),
            pl.BlockSpec((BG, E), lambda i: (i, 0)),
        ],
        out_specs=pl.BlockSpec((BG, H), lambda i: (i, 0)),
        out_shape=jax.ShapeDtypeStruct((N, H), jnp.float32),
    )(yg0, yg1, w01)


def _pack32(a):
    return jax.lax.bitcast_convert_type(
        a.reshape(*a.shape[:-1], a.shape[-1] // 2, 2), jnp.int32)


def _unpack32(a):
    return jax.lax.bitcast_convert_type(a, jnp.bfloat16).reshape(
        *a.shape[:-1], a.shape[-1] * 2)


def kernel(x, Wg, bg, W1, b1, g1, be1, W2, b2):
    d0m, d1m, w01, eidm = _gate(x, Wg, bg)
    d0 = d0m.reshape(N)
    d1 = d1m.reshape(N)
    xb16 = x.astype(jnp.bfloat16)
    xr = _unpack32(_sc_dispatch(_pack32(xb16), d0, d1))
    w1b = W1.astype(jnp.bfloat16)
    w2b = W2.astype(jnp.bfloat16)
    eid = eidm.reshape(NBLK)
    y = _expert(eid, xr, w1b, b1, g1, be1, w2b, b2)
    yg0, yg1 = _sc_gather(_pack32(y), d0, d1)
    return _combine(_unpack32(yg0), _unpack32(yg1), w01)


# speedup vs baseline: 3.3787x; 3.3787x over previous
import functools

import jax
import jax.numpy as jnp
from jax.experimental import pallas as pl
from jax.experimental.pallas import tpu as pltpu
from jax.experimental.pallas import tpu_sc as plsc

N = 2048
D = 768
H = 768
E = 8

BE = 256
PADN = 6144
NBLK = PADN // BE
BG = 256
NBG = N // BG
NC, NS = 2, 16
NW = NC * NS
BPW = N // NW


def _top2(logits):
    neg_inf = jnp.float32(-jnp.inf)
    iota = jax.lax.broadcasted_iota(jnp.int32, logits.shape, 1)
    m1 = jnp.max(logits, axis=-1, keepdims=True)
    eq1 = logits == m1
    i1 = jnp.min(jnp.where(eq1, iota, E), axis=-1, keepdims=True)
    first1 = iota == i1
    l2 = jnp.where(first1, neg_inf, logits)
    m2 = jnp.max(l2, axis=-1, keepdims=True)
    eq2 = l2 == m2
    i2 = jnp.min(jnp.where(eq2, iota, E), axis=-1, keepdims=True)
    first2 = iota == i2
    sel = first1 | first2
    wsel = jnp.where(sel, jnp.exp(logits - m1), 0.0)
    wsum = jnp.sum(wsel, axis=-1, keepdims=True)
    return first1, first2, sel, wsel, wsum


def _gate_body(x_ref, wg_ref, bg_ref, d0_ref, d1_ref, w01_ref, eid_ref,
               counts_s, po_s, run_s):
    p = pl.program_id(0)
    i = pl.program_id(1)
    xb = x_ref[...]
    logits = jnp.dot(xb, wg_ref[...], preferred_element_type=jnp.float32)
    logits = logits + bg_ref[...]
    first1, first2, sel, wsel, wsum = _top2(logits)
    sel_f = sel.astype(jnp.float32)

    @pl.when((p == 0) & (i == 0))
    def _():
        counts_s[...] = jnp.zeros((1, E), jnp.float32)

    @pl.when(p == 0)
    def _():
        counts_s[...] += jnp.sum(sel_f, axis=0, keepdims=True)

    @pl.when((p == 1) & (i == 0))
    def _():
        c = counts_s[...]
        cpad = jnp.floor((c + (BE - 1)) / BE) * BE
        lt = (jax.lax.broadcasted_iota(jnp.int32, (E, E), 0)
              < jax.lax.broadcasted_iota(jnp.int32, (E, E), 1)
              ).astype(jnp.bfloat16)
        po_s[...] = jnp.dot(cpad.astype(jnp.bfloat16), lt,
                            preferred_element_type=jnp.float32)
        run_s[...] = jnp.zeros((1, E), jnp.float32)

    @pl.when(p == 1)
    def _():
        po = po_s[...]
        run = run_s[...]
        ltn = (jax.lax.broadcasted_iota(jnp.int32, (BG, BG), 1)
               < jax.lax.broadcasted_iota(jnp.int32, (BG, BG), 0)
               ).astype(jnp.bfloat16)
        pos = run + jnp.dot(ltn, sel_f.astype(jnp.bfloat16),
                            preferred_element_type=jnp.float32)
        run_s[...] = run + jnp.sum(sel_f, axis=0, keepdims=True)
        dest = po + pos
        d0 = jnp.sum(jnp.where(first1, dest, 0.0), axis=-1, keepdims=True)
        d1 = jnp.sum(jnp.where(first2, dest, 0.0), axis=-1, keepdims=True)
        w0 = jnp.sum(jnp.where(first1, wsel, 0.0), axis=-1, keepdims=True) / wsum
        w1 = jnp.sum(jnp.where(first2, wsel, 0.0), axis=-1, keepdims=True) / wsum
        col = jax.lax.broadcasted_iota(jnp.int32, (BG, E), 1)
        d0_ref[...] = jnp.reshape(d0.astype(jnp.int32), (1, BG))
        d1_ref[...] = jnp.reshape(d1.astype(jnp.int32), (1, BG))
        w01_ref[...] = (jnp.where(col == 0, w0, 0.0)
                        + jnp.where(col == 1, w1, 0.0))
        bst = jnp.float32(BE) * jax.lax.broadcasted_iota(
            jnp.int32, (E, NBLK), 1).astype(jnp.float32)
        m = (jnp.reshape(po, (E, 1)) <= bst).astype(jnp.int32)
        eid_ref[...] = jnp.sum(m, axis=0, keepdims=True) - 1


def _gate(x, Wg, bg):
    return pl.pallas_call(
        _gate_body,
        grid=(2, NBG),
        in_specs=[
            pl.BlockSpec((BG, D), lambda p, i: (i, 0)),
            pl.BlockSpec((D, E), lambda p, i: (0, 0)),
            pl.BlockSpec((1, E), lambda p, i: (0, 0)),
        ],
        out_specs=[
            pl.BlockSpec((1, BG), lambda p, i: (0, i)),
            pl.BlockSpec((1, BG), lambda p, i: (0, i)),
            pl.BlockSpec((BG, E), lambda p, i: (i, 0)),
            pl.BlockSpec((1, NBLK), lambda p, i: (0, 0)),
        ],
        out_shape=[
            jax.ShapeDtypeStruct((1, N), jnp.int32),
            jax.ShapeDtypeStruct((1, N), jnp.int32),
            jax.ShapeDtypeStruct((N, E), jnp.float32),
            jax.ShapeDtypeStruct((1, NBLK), jnp.int32),
        ],
        scratch_shapes=[
            pltpu.VMEM((1, E), jnp.float32),
            pltpu.VMEM((1, E), jnp.float32),
            pltpu.VMEM((1, E), jnp.float32),
        ],
    )(x, Wg, bg.reshape(1, E))


def _sc_dispatch(x, d0, d1):
    mesh = plsc.VectorSubcoreMesh(core_axis_name="c", subcore_axis_name="s")

    @functools.partial(
        pl.kernel, mesh=mesh,
        out_type=jax.ShapeDtypeStruct((PADN, D), jnp.float32),
        scratch_types=[
            pltpu.VMEM((BPW,), jnp.int32),
            pltpu.VMEM((BPW,), jnp.int32),
            pltpu.VMEM((BPW, D), jnp.float32),
            pltpu.SemaphoreType.DMA,
        ],
    )
    def disp(x_hbm, d0_hbm, d1_hbm, xr_hbm, i0_v, i1_v, rows_v, sem):
        wid = jax.lax.axis_index("s") * NC + jax.lax.axis_index("c")
        base = wid * BPW
        pltpu.sync_copy(d0_hbm.at[pl.ds(base, BPW)], i0_v)
        pltpu.sync_copy(d1_hbm.at[pl.ds(base, BPW)], i1_v)
        pltpu.sync_copy(x_hbm.at[pl.ds(base, BPW)], rows_v)
        pltpu.async_copy(rows_v, xr_hbm.at[i0_v], sem).wait()
        pltpu.async_copy(rows_v, xr_hbm.at[i1_v], sem).wait()

    return disp(x, d0, d1)


def _sc_gather(y, d0, d1):
    mesh = plsc.VectorSubcoreMesh(core_axis_name="c", subcore_axis_name="s")

    @functools.partial(
        pl.kernel, mesh=mesh,
        out_type=[jax.ShapeDtypeStruct((N, H), jnp.float32),
                  jax.ShapeDtypeStruct((N, H), jnp.float32)],
        scratch_types=[
            pltpu.VMEM((BPW,), jnp.int32),
            pltpu.VMEM((BPW, H), jnp.float32),
            pltpu.SemaphoreType.DMA,
        ],
    )
    def gath(y_hbm, d0_hbm, d1_hbm, o0_hbm, o1_hbm, i_v, rows_v, sem):
        wid = jax.lax.axis_index("s") * NC + jax.lax.axis_index("c")
        base = wid * BPW
        pltpu.sync_copy(d0_hbm.at[pl.ds(base, BPW)], i_v)
        pltpu.async_copy(y_hbm.at[i_v], rows_v, sem).wait()
        pltpu.sync_copy(rows_v, o0_hbm.at[pl.ds(base, BPW)])
        pltpu.sync_copy(d1_hbm.at[pl.ds(base, BPW)], i_v)
        pltpu.async_copy(y_hbm.at[i_v], rows_v, sem).wait()
        pltpu.sync_copy(rows_v, o1_hbm.at[pl.ds(base, BPW)])

    return gath(y, d0, d1)


def _expert_body(eid_ref, xr_ref, w1_ref, b1_ref, g1_ref, be1_ref,
                 w2_ref, b2_ref, y_ref):
    xb16 = xr_ref[...].astype(jnp.bfloat16)
    h = jnp.dot(xb16, w1_ref[0], preferred_element_type=jnp.float32)
    h = h + b1_ref[0]
    mu = jnp.mean(h, axis=-1, keepdims=True)
    var = jnp.mean((h - mu) ** 2, axis=-1, keepdims=True)
    h = (h - mu) * jax.lax.rsqrt(var + 1e-5)
    h = h * g1_ref[0] + be1_ref[0]
    h = h * 0.5 * (1.0 + jax.lax.erf(h * jnp.float32(0.7071067811865476)))
    y = jnp.dot(h.astype(jnp.bfloat16), w2_ref[0],
                preferred_element_type=jnp.float32)
    y_ref[...] = y + b2_ref[0]


def _expert(eid, xr, w1b, b1, g1, be1, w2b, b2):
    grid_spec = pltpu.PrefetchScalarGridSpec(
        num_scalar_prefetch=1,
        grid=(NBLK,),
        in_specs=[
            pl.BlockSpec((BE, D), lambda i, eid_ref: (i, 0)),
            pl.BlockSpec((1, D, H), lambda i, eid_ref: (eid_ref[i], 0, 0)),
            pl.BlockSpec((1, 1, H), lambda i, eid_ref: (eid_ref[i], 0, 0)),
            pl.BlockSpec((1, 1, H), lambda i, eid_ref: (eid_ref[i], 0, 0)),
            pl.BlockSpec((1, 1, H), lambda i, eid_ref: (eid_ref[i], 0, 0)),
            pl.BlockSpec((1, D, H), lambda i, eid_ref: (eid_ref[i], 0, 0)),
            pl.BlockSpec((1, 1, H), lambda i, eid_ref: (eid_ref[i], 0, 0)),
        ],
        out_specs=pl.BlockSpec((BE, H), lambda i, eid_ref: (i, 0)),
    )
    return pl.pallas_call(
        _expert_body,
        grid_spec=grid_spec,
        out_shape=jax.ShapeDtypeStruct((PADN, H), jnp.float32),
    )(eid, xr, w1b, b1.reshape(E, 1, H), g1.reshape(E, 1, H),
      be1.reshape(E, 1, H), w2b, b2.reshape(E, 1, H))


def _combine_body(yg0_ref, yg1_ref, w01_ref, out_ref):
    w = w01_ref[...]
    out_ref[...] = yg0_ref[...] * w[:, 0:1] + yg1_ref[...] * w[:, 1:2]


def _combine(yg0, yg1, w01):
    return pl.pallas_call(
        _combine_body,
        grid=(NBG,),
        in_specs=[
            pl.BlockSpec((BG, H), lambda i: (i, 0)),
            pl.BlockSpec((BG, H), lambda i: (i, 0)),
            pl.BlockSpec((BG, E), lambda i: (i, 0)),
        ],
        out_specs=pl.BlockSpec((BG, H), lambda i: (i, 0)),
        out_shape=jax.ShapeDtypeStruct((N, H), jnp.float32),
    )(yg0, yg1, w01)


def kernel(x, Wg, bg, W1, b1, g1, be1, W2, b2):
    d0m, d1m, w01, eidm = _gate(x, Wg, bg)
    d0 = d0m.reshape(N)
    d1 = d1m.reshape(N)
    xr = _sc_dispatch(x, d0, d1)
    w1b = W1.astype(jnp.bfloat16)
    w2b = W2.astype(jnp.bfloat16)
    eid = eidm.reshape(NBLK)
    y = _expert(eid, xr, w1b, b1, g1, be1, w2b, b2)
    yg0, yg1 = _sc_gather(y, d0, d1)
    return _combine(yg0, yg1, w01)


# speedup vs baseline: 3.6701x; 1.0862x over previous
import functools

import jax
import jax.numpy as jnp
from jax.experimental import pallas as pl
from jax.experimental.pallas import tpu as pltpu
from jax.experimental.pallas import tpu_sc as plsc

N = 2048
D = 768
H = 768
E = 8

BE = 256
PADN = 6144
NBLK = PADN // BE
BG = 256
NBG = N // BG
NC, NS = 2, 16
NW = NC * NS
BPW = N // NW


def _top2(logits):
    neg_inf = jnp.float32(-jnp.inf)
    iota = jax.lax.broadcasted_iota(jnp.int32, logits.shape, 1)
    m1 = jnp.max(logits, axis=-1, keepdims=True)
    eq1 = logits == m1
    i1 = jnp.min(jnp.where(eq1, iota, E), axis=-1, keepdims=True)
    first1 = iota == i1
    l2 = jnp.where(first1, neg_inf, logits)
    m2 = jnp.max(l2, axis=-1, keepdims=True)
    eq2 = l2 == m2
    i2 = jnp.min(jnp.where(eq2, iota, E), axis=-1, keepdims=True)
    first2 = iota == i2
    sel = first1 | first2
    wsel = jnp.where(sel, jnp.exp(logits - m1), 0.0)
    wsum = jnp.sum(wsel, axis=-1, keepdims=True)
    return first1, first2, sel, wsel, wsum


def _gate_body(x_ref, wg_ref, bg_ref, d0_ref, d1_ref, w01_ref, eid_ref,
               counts_s, po_s, run_s):
    p = pl.program_id(0)
    i = pl.program_id(1)
    xb = x_ref[...]
    logits = jnp.dot(xb, wg_ref[...], preferred_element_type=jnp.float32)
    logits = logits + bg_ref[...]
    first1, first2, sel, wsel, wsum = _top2(logits)
    sel_f = sel.astype(jnp.float32)

    @pl.when((p == 0) & (i == 0))
    def _():
        counts_s[...] = jnp.zeros((1, E), jnp.float32)

    @pl.when(p == 0)
    def _():
        counts_s[...] += jnp.sum(sel_f, axis=0, keepdims=True)

    @pl.when((p == 1) & (i == 0))
    def _():
        c = counts_s[...]
        cpad = jnp.floor((c + (BE - 1)) / BE) * BE
        lt = (jax.lax.broadcasted_iota(jnp.int32, (E, E), 0)
              < jax.lax.broadcasted_iota(jnp.int32, (E, E), 1)
              ).astype(jnp.bfloat16)
        po_s[...] = jnp.dot(cpad.astype(jnp.bfloat16), lt,
                            preferred_element_type=jnp.float32)
        run_s[...] = jnp.zeros((1, E), jnp.float32)

    @pl.when(p == 1)
    def _():
        po = po_s[...]
        run = run_s[...]
        ltn = (jax.lax.broadcasted_iota(jnp.int32, (BG, BG), 1)
               < jax.lax.broadcasted_iota(jnp.int32, (BG, BG), 0)
               ).astype(jnp.bfloat16)
        pos = run + jnp.dot(ltn, sel_f.astype(jnp.bfloat16),
                            preferred_element_type=jnp.float32)
        run_s[...] = run + jnp.sum(sel_f, axis=0, keepdims=True)
        dest = po + pos
        d0 = jnp.sum(jnp.where(first1, dest, 0.0), axis=-1, keepdims=True)
        d1 = jnp.sum(jnp.where(first2, dest, 0.0), axis=-1, keepdims=True)
        w0 = jnp.sum(jnp.where(first1, wsel, 0.0), axis=-1, keepdims=True) / wsum
        w1 = jnp.sum(jnp.where(first2, wsel, 0.0), axis=-1, keepdims=True) / wsum
        col = jax.lax.broadcasted_iota(jnp.int32, (BG, E), 1)
        d0_ref[...] = jnp.reshape(d0.astype(jnp.int32), (1, BG))
        d1_ref[...] = jnp.reshape(d1.astype(jnp.int32), (1, BG))
        w01_ref[...] = (jnp.where(col == 0, w0, 0.0)
                        + jnp.where(col == 1, w1, 0.0))
        bst = jnp.float32(BE) * jax.lax.broadcasted_iota(
            jnp.int32, (E, NBLK), 1).astype(jnp.float32)
        m = (jnp.reshape(po, (E, 1)) <= bst).astype(jnp.int32)
        eid_ref[...] = jnp.sum(m, axis=0, keepdims=True) - 1


def _gate(x, Wg, bg):
    return pl.pallas_call(
        _gate_body,
        grid=(2, NBG),
        in_specs=[
            pl.BlockSpec((BG, D), lambda p, i: (i, 0)),
            pl.BlockSpec((D, E), lambda p, i: (0, 0)),
            pl.BlockSpec((1, E), lambda p, i: (0, 0)),
        ],
        out_specs=[
            pl.BlockSpec((1, BG), lambda p, i: (0, i)),
            pl.BlockSpec((1, BG), lambda p, i: (0, i)),
            pl.BlockSpec((BG, E), lambda p, i: (i, 0)),
            pl.BlockSpec((1, NBLK), lambda p, i: (0, 0)),
        ],
        out_shape=[
            jax.ShapeDtypeStruct((1, N), jnp.int32),
            jax.ShapeDtypeStruct((1, N), jnp.int32),
            jax.ShapeDtypeStruct((N, E), jnp.float32),
            jax.ShapeDtypeStruct((1, NBLK), jnp.int32),
        ],
        scratch_shapes=[
            pltpu.VMEM((1, E), jnp.float32),
            pltpu.VMEM((1, E), jnp.float32),
            pltpu.VMEM((1, E), jnp.float32),
        ],
    )(x, Wg, bg.reshape(1, E))


def _sc_dispatch(x, d0, d1):
    mesh = plsc.VectorSubcoreMesh(core_axis_name="c", subcore_axis_name="s")

    @functools.partial(
        pl.kernel, mesh=mesh,
        out_type=jax.ShapeDtypeStruct((PADN, D), jnp.float32),
        scratch_types=[
            pltpu.VMEM((BPW,), jnp.int32),
            pltpu.VMEM((BPW,), jnp.int32),
            pltpu.VMEM((BPW, D), jnp.float32),
            pltpu.SemaphoreType.DMA,
        ],
    )
    def disp(x_hbm, d0_hbm, d1_hbm, xr_hbm, i0_v, i1_v, rows_v, sem):
        wid = jax.lax.axis_index("s") * NC + jax.lax.axis_index("c")
        base = wid * BPW
        pltpu.sync_copy(d0_hbm.at[pl.ds(base, BPW)], i0_v)
        pltpu.sync_copy(d1_hbm.at[pl.ds(base, BPW)], i1_v)
        pltpu.sync_copy(x_hbm.at[pl.ds(base, BPW)], rows_v)
        pltpu.async_copy(rows_v, xr_hbm.at[i0_v], sem).wait()
        pltpu.async_copy(rows_v, xr_hbm.at[i1_v], sem).wait()

    return disp(x, d0, d1)


def _sc_gather(y, d0, d1):
    mesh = plsc.VectorSubcoreMesh(core_axis_name="c", subcore_axis_name="s")

    @functools.partial(
        pl.kernel, mesh=mesh,
        out_type=[jax.ShapeDtypeStruct((N, H), jnp.float32),
                  jax.ShapeDtypeStruct((N, H), jnp.float32)],
        scratch_types=[
            pltpu.VMEM((BPW,), jnp.int32),
            pltpu.VMEM((BPW, H), jnp.float32),
            pltpu.SemaphoreType.DMA,
        ],
    )
    def gath(y_hbm, d0_hbm, d1_hbm, o0_hbm, o1_hbm, i_v, rows_v, sem):
        wid = jax.lax.axis_index("s") * NC + jax.lax.axis_index("c")
        base = wid * BPW
        pltpu.sync_copy(d0_hbm.at[pl.ds(base, BPW)], i_v)
        pltpu.async_copy(y_hbm.at[i_v], rows_v, sem).wait()
        pltpu.sync_copy(rows_v, o0_hbm.at[pl.ds(base, BPW)])
        pltpu.sync_copy(d1_hbm.at[pl.ds(base, BPW)], i_v)
        pltpu.async_copy(y_hbm.at[i_v], rows_v, sem).wait()
        pltpu.sync_copy(rows_v, o1_hbm.at[pl.ds(base, BPW)])

    return gath(y, d0, d1)


def _expert_body(eid_ref, xr_ref, w1_ref, b1_ref, g1_ref, be1_ref,
                 w2_ref, b2_ref, y_ref, w1s, w2s):
    i = pl.program_id(0)
    prev = eid_ref[jnp.maximum(i - 1, 0)]

    @pl.when((i == 0) | (eid_ref[i] != prev))
    def _():
        w1s[...] = w1_ref[0].astype(jnp.bfloat16)
        w2s[...] = w2_ref[0].astype(jnp.bfloat16)

    xb16 = xr_ref[...].astype(jnp.bfloat16)
    h = jnp.dot(xb16, w1s[...], preferred_element_type=jnp.float32)
    h = h + b1_ref[0]
    mu = jnp.mean(h, axis=-1, keepdims=True)
    var = jnp.mean((h - mu) ** 2, axis=-1, keepdims=True)
    h = (h - mu) * jax.lax.rsqrt(var + 1e-5)
    h = h * g1_ref[0] + be1_ref[0]
    h = h * 0.5 * (1.0 + jax.lax.erf(h * jnp.float32(0.7071067811865476)))
    y = jnp.dot(h.astype(jnp.bfloat16), w2s[...],
                preferred_element_type=jnp.float32)
    y_ref[...] = y + b2_ref[0]


def _expert(eid, xr, w1b, b1, g1, be1, w2b, b2):
    grid_spec = pltpu.PrefetchScalarGridSpec(
        num_scalar_prefetch=1,
        grid=(NBLK,),
        in_specs=[
            pl.BlockSpec((BE, D), lambda i, eid_ref: (i, 0)),
            pl.BlockSpec((1, D, H), lambda i, eid_ref: (eid_ref[i], 0, 0)),
            pl.BlockSpec((1, 1, H), lambda i, eid_ref: (eid_ref[i], 0, 0)),
            pl.BlockSpec((1, 1, H), lambda i, eid_ref: (eid_ref[i], 0, 0)),
            pl.BlockSpec((1, 1, H), lambda i, eid_ref: (eid_ref[i], 0, 0)),
            pl.BlockSpec((1, D, H), lambda i, eid_ref: (eid_ref[i], 0, 0)),
            pl.BlockSpec((1, 1, H), lambda i, eid_ref: (eid_ref[i], 0, 0)),
        ],
        out_specs=pl.BlockSpec((BE, H), lambda i, eid_ref: (i, 0)),
        scratch_shapes=[
            pltpu.VMEM((D, H), jnp.bfloat16),
            pltpu.VMEM((H, H), jnp.bfloat16),
        ],
    )
    return pl.pallas_call(
        _expert_body,
        grid_spec=grid_spec,
        out_shape=jax.ShapeDtypeStruct((PADN, H), jnp.float32),
    )(eid, xr, w1b, b1.reshape(E, 1, H), g1.reshape(E, 1, H),
      be1.reshape(E, 1, H), w2b, b2.reshape(E, 1, H))


def _combine_body(yg0_ref, yg1_ref, w01_ref, out_ref):
    w = w01_ref[...]
    out_ref[...] = yg0_ref[...] * w[:, 0:1] + yg1_ref[...] * w[:, 1:2]


def _combine(yg0, yg1, w01):
    return pl.pallas_call(
        _combine_body,
        grid=(NBG,),
        in_specs=[
            pl.BlockSpec((BG, H), lambda i: (i, 0)),
            pl.BlockSpec((BG, H), lambda i: (i, 0)),
            pl.BlockSpec((BG, E), lambda i: (i, 0)),
        ],
        out_specs=pl.BlockSpec((BG, H), lambda i: (i, 0)),
        out_shape=jax.ShapeDtypeStruct((N, H), jnp.float32),
    )(yg0, yg1, w01)


def kernel(x, Wg, bg, W1, b1, g1, be1, W2, b2):
    d0m, d1m, w01, eidm = _gate(x, Wg, bg)
    d0 = d0m.reshape(N)
    d1 = d1m.reshape(N)
    xr = _sc_dispatch(x, d0, d1)
    eid = eidm.reshape(NBLK)
    y = _expert(eid, xr, W1, b1, g1, be1, W2, b2)
    yg0, yg1 = _sc_gather(y, d0, d1)
    return _combine(yg0, yg1, w01)


# speedup vs baseline: 5.5692x; 1.5175x over previous
import functools

import jax
import jax.numpy as jnp
from jax.experimental import pallas as pl

N = 2048
D = 768
H = 768
E = 8
BN = 1024


def _moe_body(x_ref, wg_ref, bg_ref, w1_ref, b1_ref, g1_ref, be1_ref,
              w2_ref, b2_ref, out_ref):
    xb = x_ref[...]

    logits = jnp.dot(xb, wg_ref[...], preferred_element_type=jnp.float32)
    logits = logits + bg_ref[...]
    neg_inf = jnp.float32(-jnp.inf)
    iota = jax.lax.broadcasted_iota(jnp.int32, logits.shape, 1)
    m1 = jnp.max(logits, axis=-1, keepdims=True)
    eq1 = logits == m1
    i1 = jnp.min(jnp.where(eq1, iota, E), axis=-1, keepdims=True)
    first1 = iota == i1
    l2 = jnp.where(first1, neg_inf, logits)
    m2 = jnp.max(l2, axis=-1, keepdims=True)
    eq2 = l2 == m2
    i2 = jnp.min(jnp.where(eq2, iota, E), axis=-1, keepdims=True)
    first2 = iota == i2
    sel = first1 | first2
    wsel = jnp.where(sel, jnp.exp(logits - m1), 0.0)
    cw = wsel / jnp.sum(wsel, axis=-1, keepdims=True)

    xb16 = xb.astype(jnp.bfloat16)
    parts = []
    for e in range(E):
        h = jnp.dot(xb16, w1_ref[e], preferred_element_type=jnp.float32)
        h = h + b1_ref[e][None, :]
        mu = jnp.mean(h, axis=-1, keepdims=True)
        var = jnp.mean((h - mu) ** 2, axis=-1, keepdims=True)
        h = (h - mu) * jax.lax.rsqrt(var + 1e-5)
        h = h * g1_ref[e][None, :] + be1_ref[e][None, :]
        h = h * 0.5 * (1.0 + jax.lax.erf(h * jnp.float32(0.7071067811865476)))
        parts.append((h * cw[:, e][:, None]).astype(jnp.bfloat16))
    hcat = jnp.concatenate(parts, axis=1)
    out = jnp.dot(hcat, w2_ref[...], preferred_element_type=jnp.float32)
    out_ref[...] = out + jnp.dot(cw, b2_ref[...],
                                 preferred_element_type=jnp.float32)


def kernel(x, Wg, bg, W1, b1, g1, be1, W2, b2):
    w1b = W1.astype(jnp.bfloat16)
    w2c = W2.astype(jnp.bfloat16).reshape(E * H, H)
    grid = (N // BN,)
    const = lambda i: (0, 0)
    const3 = lambda i: (0, 0, 0)
    out = pl.pallas_call(
        _moe_body,
        grid=grid,
        in_specs=[
            pl.BlockSpec((BN, D), lambda i: (i, 0)),
            pl.BlockSpec((D, E), const),
            pl.BlockSpec((1, E), const),
            pl.BlockSpec((E, D, H), const3),
            pl.BlockSpec((E, H), const),
            pl.BlockSpec((E, H), const),
            pl.BlockSpec((E, H), const),
            pl.BlockSpec((E * H, H), const),
            pl.BlockSpec((E, H), const),
        ],
        out_specs=pl.BlockSpec((BN, H), lambda i: (i, 0)),
        out_shape=jax.ShapeDtypeStruct((N, H), jnp.float32),
    )(x, Wg, bg.reshape(1, E), w1b, b1, g1, be1, w2c, b2)
    return out


# speedup vs baseline: 6.6292x; 1.1903x over previous
import jax
import jax.numpy as jnp
from jax.experimental import pallas as pl
from jax.experimental.pallas import tpu as pltpu

N = 2048
D = 768
H = 768
E = 8
BN = 512
NB = N // BN


def _moe_body(x_ref, wg_ref, bg_ref, w1_ref, b1_ref, g1_ref, be1_ref,
              w2_ref, b2_ref, out_ref, w1s, w2s):
    s = pl.program_id(0)

    @pl.when(s < E)
    def _():
        e = jnp.minimum(s, E - 1)
        w1s[e] = w1_ref[0].astype(jnp.bfloat16)
        w2s[pl.ds(e * H, H), :] = w2_ref[0].astype(jnp.bfloat16)

    @pl.when(s >= E)
    def _():
        xb = x_ref[...]

        logits = jnp.dot(xb, wg_ref[...], preferred_element_type=jnp.float32)
        logits = logits + bg_ref[...]
        neg_inf = jnp.float32(-jnp.inf)
        iota = jax.lax.broadcasted_iota(jnp.int32, logits.shape, 1)
        m1 = jnp.max(logits, axis=-1, keepdims=True)
        eq1 = logits == m1
        i1 = jnp.min(jnp.where(eq1, iota, E), axis=-1, keepdims=True)
        first1 = iota == i1
        l2 = jnp.where(first1, neg_inf, logits)
        m2 = jnp.max(l2, axis=-1, keepdims=True)
        eq2 = l2 == m2
        i2 = jnp.min(jnp.where(eq2, iota, E), axis=-1, keepdims=True)
        first2 = iota == i2
        sel = first1 | first2
        wsel = jnp.where(sel, jnp.exp(logits - m1), 0.0)
        cw = wsel / jnp.sum(wsel, axis=-1, keepdims=True)

        xb16 = xb.astype(jnp.bfloat16)
        parts = []
        for e in range(E):
            h = jnp.dot(xb16, w1s[e], preferred_element_type=jnp.float32)
            h = h + b1_ref[e][None, :]
            mu = jnp.mean(h, axis=-1, keepdims=True)
            var = jnp.mean((h - mu) ** 2, axis=-1, keepdims=True)
            h = (h - mu) * jax.lax.rsqrt(var + 1e-5)
            h = h * g1_ref[e][None, :] + be1_ref[e][None, :]
            h = h * 0.5 * (1.0 + jax.lax.erf(h * jnp.float32(0.7071067811865476)))
            parts.append((h * cw[:, e][:, None]).astype(jnp.bfloat16))
        hcat = jnp.concatenate(parts, axis=1)
        out = jnp.dot(hcat, w2s[...], preferred_element_type=jnp.float32)
        out_ref[...] = out + jnp.dot(cw, b2_ref[...],
                                     preferred_element_type=jnp.float32)


def kernel(x, Wg, bg, W1, b1, g1, be1, W2, b2):
    const = lambda s: (0, 0)

    def wmap(s):
        return (jnp.minimum(s, E - 1), 0, 0)

    def xmap(s):
        return (jnp.maximum(s - E, 0), 0)

    out = pl.pallas_call(
        _moe_body,
        grid=(E + NB,),
        in_specs=[
            pl.BlockSpec((BN, D), xmap),
            pl.BlockSpec((D, E), const),
            pl.BlockSpec((1, E), const),
            pl.BlockSpec((1, D, H), wmap),
            pl.BlockSpec((E, H), const),
            pl.BlockSpec((E, H), const),
            pl.BlockSpec((E, H), const),
            pl.BlockSpec((1, H, H), wmap),
            pl.BlockSpec((E, H), const),
        ],
        out_specs=pl.BlockSpec((BN, H), xmap),
        out_shape=jax.ShapeDtypeStruct((N, H), jnp.float32),
        scratch_shapes=[
            pltpu.VMEM((E, D, H), jnp.bfloat16),
            pltpu.VMEM((E * H, H), jnp.bfloat16),
        ],
    )(x, Wg, bg.reshape(1, E), W1, b1, g1, be1, W2, b2)
    return out
